# initial kernel scaffold (unmeasured)
import jax
import jax.numpy as jnp
from jax import lax
from jax.experimental import pallas as pl
from jax.experimental.pallas import tpu as pltpu

N_DEV = 8
SQ = 1024
DM = 1024
NH = 64
HL = 8
DH = 128
HDL = HL * DH
SKVL = 1024
NC = 4
RC = 256
SCALE = 0.08838834764831843
F32 = jnp.float32
BF16 = jnp.bfloat16
NSLOT = 4


def _class_group(v):
    n = v.shape[-1]
    v = v.reshape(4, 4, 64, n)
    v = jnp.transpose(v, (1, 0, 2, 3))
    return v.reshape(NC, RC, n)


def kernel(x, Wq, K_ext, V_ext, Wo):
    x2 = x.reshape(SQ, DM)
    k2 = K_ext.reshape(SKVL, NH * DH)
    v2 = V_ext.reshape(SKVL, NH * DH)

    def body(x_ref, wq_ref, k_ref, v_ref, wo_ref, out_ref,
             tmpf, xb, wqb, wob, qcg, sendbuf, krecv, vrecv, arbuf, redbuf,
             copy_sem, ssend, ksem, vsem, arsend, arrecv, agsend, agrecv):
        me = lax.axis_index("i")

        barrier = pltpu.get_barrier_semaphore()
        for d in range(1, N_DEV):
            pl.semaphore_signal(barrier, inc=1,
                                device_id=((me + d) % N_DEV,),
                                device_id_type=pl.DeviceIdType.MESH)
        pl.semaphore_wait(barrier, N_DEV - 1)

        def pull(src):
            cp = pltpu.make_async_copy(src, tmpf, copy_sem)
            cp.start()
            cp.wait()

        def stage(src_ref, j, slot):
            pull(src_ref.at[:, pl.ds(j * HDL, HDL)])
            sendbuf[slot] = _class_group(tmpf[...].astype(BF16))

        for src_ref, dst, slot in ((k_ref, krecv, 0), (v_ref, vrecv, 1)):
            stage(src_ref, me, slot)
            cp = pltpu.make_async_copy(sendbuf.at[slot], dst.at[me], copy_sem)
            cp.start()
            cp.wait()

        descs = []
        seq = 0
        for d in range(1, N_DEV):
            j = (me + d) % N_DEV
            for src_ref, dst, sems in ((k_ref, krecv, ksem),
                                       (v_ref, vrecv, vsem)):
                slot = seq % NSLOT
                if seq >= NSLOT:
                    descs[seq - NSLOT].wait_send()
                stage(src_ref, j, slot)
                rdma = pltpu.make_async_remote_copy(
                    src_ref=sendbuf.at[slot],
                    dst_ref=dst.at[me],
                    send_sem=ssend.at[slot],
                    recv_sem=sems.at[me],
                    device_id=(j,),
                    device_id_type=pl.DeviceIdType.MESH,
                )
                rdma.start()
                descs.append(rdma)
                seq += 1

        pull(x_ref)
        xb[...] = tmpf[...].astype(BF16)
        pull(wq_ref)
        wqb[...] = tmpf[...].astype(BF16)
        q = jnp.dot(xb[...], wqb[...], preferred_element_type=F32)
        qcg[...] = _class_group(q.astype(BF16))
        pull(wo_ref)
        wob[...] = tmpf[...].astype(BF16)

        for s in range(max(0, seq - NSLOT), seq):
            descs[s].wait_send()

        for d in range(1, N_DEV):
            j = (me + d) % N_DEV
            for dst, sems in ((krecv, ksem), (vrecv, vsem)):
                pltpu.make_async_remote_copy(
                    src_ref=sendbuf.at[0],
                    dst_ref=dst.at[j],
                    send_sem=ssend.at[0],
                    recv_sem=sems.at[j],
                    device_id=(j,),
                    device_id_type=pl.DeviceIdType.MESH,
                ).wait_recv()

        for c in range(NC):
            acc = None
            for h in range(HL):
                sl = slice(h * DH, (h + 1) * DH)
                qt = qcg[c, :, sl]
                kt = krecv[:, c, :, sl].reshape(N_DEV * RC, DH)
                vt = vrecv[:, c, :, sl].reshape(N_DEV * RC, DH)
                s = lax.dot_general(qt, kt, (((1,), (1,)), ((), ())),
                                    preferred_element_type=F32) * SCALE
                m = jnp.max(s, axis=1, keepdims=True)
                p = jnp.exp(s - m)
                l = jnp.sum(p, axis=1, keepdims=True)
                ctx = lax.dot_general(p.astype(BF16), vt,
                                      (((1,), (0,)), ((), ())),
                                      preferred_element_type=F32)
                ctx = ctx / l
                delta = jnp.dot(ctx.astype(BF16), wob[sl, :],
                                preferred_element_type=F32)
                acc = delta if acc is None else acc + delta
            for b in range(4):
                out_ref[0, (4 * b + c) * 64:(4 * b + c + 1) * 64, :] = \
                    acc[b * 64:(b + 1) * 64, :]

        xb[...] = out_ref[0].astype(BF16)
        ar_descs = []
        for d in range(1, N_DEV):
            j = (me + d) % N_DEV
            rdma = pltpu.make_async_remote_copy(
                src_ref=xb.at[pl.ds(j * 128, 128), :],
                dst_ref=arbuf.at[me],
                send_sem=arsend.at[j],
                recv_sem=arrecv.at[me],
                device_id=(j,),
                device_id_type=pl.DeviceIdType.MESH,
            )
            rdma.start()
            ar_descs.append(rdma)

        cp = pltpu.make_async_copy(out_ref.at[0, pl.ds(me * 128, 128), :],
                                   redbuf, copy_sem)
        cp.start()
        cp.wait()
        red = redbuf[...]
        for d in range(1, N_DEV):
            j = (me + d) % N_DEV
            pltpu.make_async_remote_copy(
                src_ref=xb.at[pl.ds(0, 128), :],
                dst_ref=arbuf.at[j],
                send_sem=arsend.at[0],
                recv_sem=arrecv.at[j],
                device_id=(j,),
                device_id_type=pl.DeviceIdType.MESH,
            ).wait_recv()
            red = red + arbuf[j].astype(F32)
        redbuf[...] = red
        cp = pltpu.make_async_copy(redbuf,
                                   out_ref.at[0, pl.ds(me * 128, 128), :],
                                   copy_sem)
        cp.start()
        cp.wait()

        ag_descs = []
        for d in range(1, N_DEV):
            j = (me + d) % N_DEV
            rdma = pltpu.make_async_remote_copy(
                src_ref=redbuf,
                dst_ref=out_ref.at[0, pl.ds(me * 128, 128), :],
                send_sem=agsend.at[j],
                recv_sem=agrecv.at[me],
                device_id=(j,),
                device_id_type=pl.DeviceIdType.MESH,
            )
            rdma.start()
            ag_descs.append(rdma)
        for rdma in ar_descs:
            rdma.wait_send()
        for d in range(1, N_DEV):
            j = (me + d) % N_DEV
            pltpu.make_async_remote_copy(
                src_ref=redbuf,
                dst_ref=out_ref.at[0, pl.ds(j * 128, 128), :],
                send_sem=agsend.at[0],
                recv_sem=agrecv.at[j],
                device_id=(j,),
                device_id_type=pl.DeviceIdType.MESH,
            ).wait_recv()
        for rdma in ag_descs:
            rdma.wait_send()

    return pl.pallas_call(
        body,
        out_shape=jax.ShapeDtypeStruct((1, SQ, DM), F32),
        in_specs=[pl.BlockSpec(memory_space=pltpu.ANY)] * 5,
        out_specs=pl.BlockSpec(memory_space=pltpu.VMEM),
        scratch_shapes=[
            pltpu.VMEM((SQ, DM), F32),
            pltpu.VMEM((SQ, DM), BF16),
            pltpu.VMEM((SQ, DM), BF16),
            pltpu.VMEM((SQ, DM), BF16),
            pltpu.VMEM((NC, RC, HDL), BF16),
            pltpu.VMEM((NSLOT, NC, RC, HDL), BF16),
            pltpu.VMEM((N_DEV, NC, RC, HDL), BF16),
            pltpu.VMEM((N_DEV, NC, RC, HDL), BF16),
            pltpu.VMEM((N_DEV, 128, DM), BF16),
            pltpu.VMEM((128, DM), F32),
            pltpu.SemaphoreType.DMA,
            pltpu.SemaphoreType.DMA((NSLOT,)),
            pltpu.SemaphoreType.DMA((N_DEV,)),
            pltpu.SemaphoreType.DMA((N_DEV,)),
            pltpu.SemaphoreType.DMA((N_DEV,)),
            pltpu.SemaphoreType.DMA((N_DEV,)),
            pltpu.SemaphoreType.DMA((N_DEV,)),
            pltpu.SemaphoreType.DMA((N_DEV,)),
        ],
        compiler_params=pltpu.CompilerParams(collective_id=0),
    )(x2, Wq, k2, v2, Wo)


# baseline (device time: 478260 ns/iter reference)
import jax
import jax.numpy as jnp
from jax import lax
from jax.experimental import pallas as pl
from jax.experimental.pallas import tpu as pltpu

N_DEV = 8
SQ = 1024
DM = 1024
NH = 64
HL = 8
DH = 128
HDL = HL * DH
SKVL = 1024
NC = 4
RC = 256
SCALE = 0.08838834764831843
F32 = jnp.float32
BF16 = jnp.bfloat16
NSLOT = 4


def _class_group(v):
    n = v.shape[-1]
    v = v.reshape(4, 4, 64, n)
    v = jnp.transpose(v, (1, 0, 2, 3))
    return v.reshape(NC, RC, n)


def kernel(x, Wq, K_ext, V_ext, Wo):
    x2 = x.reshape(SQ, DM)
    k2 = K_ext.reshape(SKVL, NH * DH)
    v2 = V_ext.reshape(SKVL, NH * DH)

    def body(x_ref, wq_ref, k_ref, v_ref, wo_ref, out_ref,
             tmpf, xb, wqb, wob, qcg, sendbuf, krecv, vrecv, arbuf, redbuf,
             copy_sem, ssend, ksem, vsem, arsend, arrecv, agsend, agrecv):
        me = lax.axis_index("i")

        barrier = pltpu.get_barrier_semaphore()
        for d in range(1, N_DEV):
            pl.semaphore_signal(barrier, inc=1,
                                device_id=((me + d) % N_DEV,),
                                device_id_type=pl.DeviceIdType.MESH)
        pl.semaphore_wait(barrier, N_DEV - 1)

        def pull(src):
            cp = pltpu.make_async_copy(src, tmpf, copy_sem)
            cp.start()
            cp.wait()

        def stage(src_ref, j, slot):
            pull(src_ref.at[:, pl.ds(j * HDL, HDL)])
            sendbuf[slot] = _class_group(tmpf[...].astype(BF16))

        for src_ref, dst, slot in ((k_ref, krecv, 0), (v_ref, vrecv, 1)):
            stage(src_ref, me, slot)
            cp = pltpu.make_async_copy(sendbuf.at[slot], dst.at[me], copy_sem)
            cp.start()
            cp.wait()

        descs = []
        seq = 0
        for d in range(1, N_DEV):
            j = (me + d) % N_DEV
            for src_ref, dst, sems in ((k_ref, krecv, ksem),
                                       (v_ref, vrecv, vsem)):
                slot = seq % NSLOT
                if seq >= NSLOT:
                    descs[seq - NSLOT].wait_send()
                stage(src_ref, j, slot)
                rdma = pltpu.make_async_remote_copy(
                    src_ref=sendbuf.at[slot],
                    dst_ref=dst.at[me],
                    send_sem=ssend.at[slot],
                    recv_sem=sems.at[me],
                    device_id=(j,),
                    device_id_type=pl.DeviceIdType.MESH,
                )
                rdma.start()
                descs.append(rdma)
                seq += 1

        pull(x_ref)
        xb[...] = tmpf[...].astype(BF16)
        pull(wq_ref)
        wqb[...] = tmpf[...].astype(BF16)
        q = jnp.dot(xb[...], wqb[...], preferred_element_type=F32)
        qcg[...] = _class_group(q.astype(BF16))
        pull(wo_ref)
        wob[...] = tmpf[...].astype(BF16)

        for s in range(max(0, seq - NSLOT), seq):
            descs[s].wait_send()

        for d in range(1, N_DEV):
            j = (me + d) % N_DEV
            for dst, sems in ((krecv, ksem), (vrecv, vsem)):
                pltpu.make_async_remote_copy(
                    src_ref=sendbuf.at[0],
                    dst_ref=dst.at[j],
                    send_sem=ssend.at[0],
                    recv_sem=sems.at[j],
                    device_id=(j,),
                    device_id_type=pl.DeviceIdType.MESH,
                ).wait_recv()

        for c in range(NC):
            acc = None
            for h in range(HL):
                sl = slice(h * DH, (h + 1) * DH)
                qt = qcg[c, :, sl]
                kt = krecv[:, c, :, sl].reshape(N_DEV * RC, DH)
                vt = vrecv[:, c, :, sl].reshape(N_DEV * RC, DH)
                s = lax.dot_general(qt, kt, (((1,), (1,)), ((), ())),
                                    preferred_element_type=F32) * SCALE
                m = jnp.max(s, axis=1, keepdims=True)
                p = jnp.exp(s - m)
                l = jnp.sum(p, axis=1, keepdims=True)
                ctx = lax.dot_general(p.astype(BF16), vt,
                                      (((1,), (0,)), ((), ())),
                                      preferred_element_type=F32)
                ctx = ctx / l
                delta = jnp.dot(ctx.astype(BF16), wob[sl, :],
                                preferred_element_type=F32)
                acc = delta if acc is None else acc + delta
            for b in range(4):
                out_ref[0, (4 * b + c) * 64:(4 * b + c + 1) * 64, :] = \
                    acc[b * 64:(b + 1) * 64, :]

        xb[...] = out_ref[0].astype(BF16)
        ar_descs = []
        for d in range(1, N_DEV):
            j = (me + d) % N_DEV
            rdma = pltpu.make_async_remote_copy(
                src_ref=xb.at[pl.ds(j * 128, 128), :],
                dst_ref=arbuf.at[me],
                send_sem=arsend.at[j],
                recv_sem=arrecv.at[me],
                device_id=(j,),
                device_id_type=pl.DeviceIdType.MESH,
            )
            rdma.start()
            ar_descs.append(rdma)

        cp = pltpu.make_async_copy(out_ref.at[0, pl.ds(me * 128, 128), :],
                                   redbuf, copy_sem)
        cp.start()
        cp.wait()
        red = redbuf[...]
        for d in range(1, N_DEV):
            j = (me + d) % N_DEV
            pltpu.make_async_remote_copy(
                src_ref=xb.at[pl.ds(0, 128), :],
                dst_ref=arbuf.at[j],
                send_sem=arsend.at[0],
                recv_sem=arrecv.at[j],
                device_id=(j,),
                device_id_type=pl.DeviceIdType.MESH,
            ).wait_recv()
            red = red + arbuf[j].astype(F32)
        redbuf[...] = red
        cp = pltpu.make_async_copy(redbuf,
                                   out_ref.at[0, pl.ds(me * 128, 128), :],
                                   copy_sem)
        cp.start()
        cp.wait()

        ag_descs = []
        for d in range(1, N_DEV):
            j = (me + d) % N_DEV
            rdma = pltpu.make_async_remote_copy(
                src_ref=redbuf,
                dst_ref=out_ref.at[0, pl.ds(me * 128, 128), :],
                send_sem=agsend.at[j],
                recv_sem=agrecv.at[me],
                device_id=(j,),
                device_id_type=pl.DeviceIdType.MESH,
            )
            rdma.start()
            ag_descs.append(rdma)
        for rdma in ar_descs:
            rdma.wait_send()
        for d in range(1, N_DEV):
            j = (me + d) % N_DEV
            pltpu.make_async_remote_copy(
                src_ref=redbuf,
                dst_ref=out_ref.at[0, pl.ds(j * 128, 128), :],
                send_sem=agsend.at[0],
                recv_sem=agrecv.at[j],
                device_id=(j,),
                device_id_type=pl.DeviceIdType.MESH,
            ).wait_recv()
        for rdma in ag_descs:
            rdma.wait_send()

    return pl.pallas_call(
        body,
        out_shape=jax.ShapeDtypeStruct((1, SQ, DM), F32),
        in_specs=[pl.BlockSpec(memory_space=pl.ANY)] * 5,
        out_specs=pl.BlockSpec(memory_space=pltpu.MemorySpace.VMEM),
        scratch_shapes=[
            pltpu.VMEM((SQ, DM), F32),
            pltpu.VMEM((SQ, DM), BF16),
            pltpu.VMEM((SQ, DM), BF16),
            pltpu.VMEM((SQ, DM), BF16),
            pltpu.VMEM((NC, RC, HDL), BF16),
            pltpu.VMEM((NSLOT, NC, RC, HDL), BF16),
            pltpu.VMEM((N_DEV, NC, RC, HDL), BF16),
            pltpu.VMEM((N_DEV, NC, RC, HDL), BF16),
            pltpu.VMEM((N_DEV, 128, DM), BF16),
            pltpu.VMEM((128, DM), F32),
            pltpu.SemaphoreType.DMA,
            pltpu.SemaphoreType.DMA((NSLOT,)),
            pltpu.SemaphoreType.DMA((N_DEV,)),
            pltpu.SemaphoreType.DMA((N_DEV,)),
            pltpu.SemaphoreType.DMA((N_DEV,)),
            pltpu.SemaphoreType.DMA((N_DEV,)),
            pltpu.SemaphoreType.DMA((N_DEV,)),
            pltpu.SemaphoreType.DMA((N_DEV,)),
        ],
        compiler_params=pltpu.CompilerParams(
            collective_id=0, vmem_limit_bytes=100 * 1024 * 1024),
    )(x2, Wq, k2, v2, Wo)
